# baseline (speedup 1.0000x reference)
import functools

import jax
import jax.numpy as jnp
from jax import lax
from jax.experimental import pallas as pl
from jax.experimental.pallas import tpu as pltpu
from jax.experimental.pallas import tpu_sc as plsc

MARGIN_ = 1.0
N_RULES = 100000
L = 16
NW = 16
CHUNK = 6272
LAST = N_RULES - (NW - 1) * CHUNK
NEG = float("-inf")
UNROLL = 4


def _hnl_body(rs_hbm, forb_hbm, out_hbm,
              chunk_v, forb_v, tgt_v, pos_v, tloc_v, cand_v, out_v,
              shared, sem, *sems):
    cid = lax.axis_index("c")
    wid = lax.axis_index("s")

    @pl.when(cid == 0)
    def _work():
        base = wid * CHUNK
        neg_vec = jnp.full((L,), NEG, jnp.float32)

        with jax.named_scope("hnl_stage"):
            forb_cp = pltpu.async_copy(forb_hbm, forb_v, sem)
            S = LAST // 4
            cps = [
                pltpu.async_copy(rs_hbm.at[pl.ds(base + j * S, S)],
                                 chunk_v.at[pl.ds(j * S, S)], sems[j])
                for j in range(4)
            ]
            tail_src = jnp.where(wid == NW - 1, 0, base + LAST)
            cp_t = pltpu.async_copy(rs_hbm.at[pl.ds(tail_src, CHUNK - LAST)],
                                    chunk_v.at[pl.ds(LAST, CHUNK - LAST)],
                                    sems[4])
            forb_cp.wait()
            tvec = forb_v[pl.ds(0, L)]

            for cp in cps:
                cp.wait()
            cp_t.wait()

            @pl.when(wid == NW - 1)
            def _pad():
                for j in range((CHUNK - LAST) // L):
                    chunk_v[pl.ds(LAST + j * L, L)] = neg_vec

        with jax.named_scope("hnl_scatter"):
            for j in range(5):
                ids = forb_v[pl.ds(j * L, L)]
                local = ids - base
                ok = (local >= 0) & (local < CHUNK)
                clamped = jnp.minimum(jnp.maximum(local, 0), CHUNK - 1)
                plsc.store_scatter(chunk_v, [clamped], neg_vec, mask=ok)

        def insert(ts, v):
            out = []
            for t in ts:
                hi = jnp.maximum(t, v)
                v = jnp.minimum(t, v)
                out.append(hi)
            return tuple(out)

        def step(i, ch):
            new = []
            for u in range(UNROLL):
                v = chunk_v[pl.ds((i * UNROLL + u) * L, L)]
                new.append(insert(ch[u], v))
            return tuple(new)

        init = tuple((neg_vec,) * 5 for _ in range(UNROLL))
        with jax.named_scope("hnl_loop"):
            chains = lax.fori_loop(0, CHUNK // (L * UNROLL), step, init)

        ts = chains[0]
        for u in range(1, UNROLL):
            for v in chains[u]:
                ts = insert(ts, v)

        s, _ = plsc.sort_key_val(ts[0], ts[0], descending=True)
        for j in range(1, 5):
            b, _ = plsc.sort_key_val(ts[j], ts[j], descending=True)
            c = jnp.maximum(s, lax.rev(b, (0,)))
            s, _ = plsc.sort_key_val(c, c, descending=True)

        @pl.when(wid == 0)
        def _pos():
            tgt_v[...] = tvec
            pltpu.async_copy(rs_hbm.at[tgt_v], pos_v, sem).wait()

        with jax.named_scope("hnl_pub"):
            tloc_v[...] = s
            pltpu.sync_copy(tloc_v, shared.at[pl.ds(wid * L, L)])
            plsc.subcore_barrier()

        @pl.when(wid == 0)
        def _merge():
          with jax.named_scope("hnl_merge"):
            pltpu.sync_copy(shared, cand_v)
            m = cand_v[pl.ds(0, L)]
            for k in range(1, NW):
                b = cand_v[pl.ds(k * L, L)]
                c = jnp.maximum(m, lax.rev(b, (0,)))
                m, _ = plsc.sort_key_val(c, c, descending=True)

            lane = lax.iota(jnp.int32, L)
            f = jnp.maximum(MARGIN_ - pos_v[...] + m, 0.0)
            sel = jnp.where(lane < 5, f, 0.0)
            loss = jnp.sum(sel) * (1.0 / 5.0)
            out_v[...] = jnp.broadcast_to(loss, (L,))
            pltpu.sync_copy(out_v, out_hbm)


_hnl = functools.partial(
    pl.kernel,
    out_type=jax.ShapeDtypeStruct((L,), jnp.float32),
    mesh=plsc.VectorSubcoreMesh(core_axis_name="c", subcore_axis_name="s",
                                num_cores=1),
    compiler_params=pltpu.CompilerParams(needs_layout_passes=False,
                                         disable_bounds_checks=True,
                                         disable_semaphore_checks=True),
    scratch_types=[
        pltpu.VMEM((CHUNK,), jnp.float32),
        pltpu.VMEM((80,), jnp.int32),
        pltpu.VMEM((L,), jnp.int32),
        pltpu.VMEM((L,), jnp.float32),
        pltpu.VMEM((L,), jnp.float32),
        pltpu.VMEM((NW * L,), jnp.float32),
        pltpu.VMEM((L,), jnp.float32),
        pltpu.VMEM_SHARED((NW * L,), jnp.float32),
        pltpu.SemaphoreType.DMA,
        pltpu.SemaphoreType.DMA,
        pltpu.SemaphoreType.DMA,
        pltpu.SemaphoreType.DMA,
        pltpu.SemaphoreType.DMA,
        pltpu.SemaphoreType.DMA,
    ],
)(_hnl_body)


def kernel(rule_scores, target_rule, available_rules):
    tgt = jnp.asarray(target_rule, jnp.int32)
    forb = jnp.concatenate(
        [jnp.broadcast_to(tgt, (L,)), available_rules.astype(jnp.int32)])
    out16 = _hnl(rule_scores, forb)
    return out16[0]


# speedup vs baseline: 1.0137x; 1.0137x over previous
import functools

import jax
import jax.numpy as jnp
from jax import lax
from jax.experimental import pallas as pl
from jax.experimental.pallas import tpu as pltpu
from jax.experimental.pallas import tpu_sc as plsc

MARGIN_ = 1.0
N_RULES = 100000
L = 16
NW = 16
CHUNK = 6272
LAST = N_RULES - (NW - 1) * CHUNK
NEG = float("-inf")
UNROLL = 4


def _hnl_body(rs_hbm, forb_hbm, out_hbm,
              chunk_v, forb_v, tgt_v, pos_v, tloc_v, cand_v, out_v,
              shared, sem, *sems):
    cid = lax.axis_index("c")
    wid = lax.axis_index("s")

    @pl.when(cid == 0)
    def _work():
        base = wid * CHUNK
        neg_vec = jnp.full((L,), NEG, jnp.float32)

        with jax.named_scope("hnl_stage"):
            S = LAST // 4
            cps = [
                pltpu.async_copy(rs_hbm.at[pl.ds(base + j * S, S)],
                                 chunk_v.at[pl.ds(j * S, S)], sems[j])
                for j in range(4)
            ]
            tail_src = jnp.where(wid == NW - 1, 0, base + LAST)
            cp_t = pltpu.async_copy(rs_hbm.at[pl.ds(tail_src, CHUNK - LAST)],
                                    chunk_v.at[pl.ds(LAST, CHUNK - LAST)],
                                    sems[4])
            pltpu.sync_copy(forb_hbm, forb_v)
            tvec = forb_v[pl.ds(0, L)]

            for cp in cps:
                cp.wait()
            cp_t.wait()

            @pl.when(wid == NW - 1)
            def _pad():
                for j in range((CHUNK - LAST) // L):
                    chunk_v[pl.ds(LAST + j * L, L)] = neg_vec

        with jax.named_scope("hnl_scatter"):
            for j in range(5):
                ids = forb_v[pl.ds(j * L, L)]
                local = ids - base
                ok = (local >= 0) & (local < CHUNK)
                clamped = jnp.minimum(jnp.maximum(local, 0), CHUNK - 1)
                plsc.store_scatter(chunk_v, [clamped], neg_vec, mask=ok)

        def insert(ts, v):
            out = []
            for t in ts:
                hi = jnp.maximum(t, v)
                v = jnp.minimum(t, v)
                out.append(hi)
            return tuple(out)

        def step(i, ch):
            new = []
            for u in range(UNROLL):
                v = chunk_v[pl.ds((i * UNROLL + u) * L, L)]
                new.append(insert(ch[u], v))
            return tuple(new)

        init = tuple((neg_vec,) * 5 for _ in range(UNROLL))
        with jax.named_scope("hnl_loop"):
            chains = lax.fori_loop(0, CHUNK // (L * UNROLL), step, init)

        ts = chains[0]
        for u in range(1, UNROLL):
            for v in chains[u]:
                ts = insert(ts, v)

        s, _ = plsc.sort_key_val(ts[0], ts[0], descending=True)
        for j in range(1, 5):
            b, _ = plsc.sort_key_val(ts[j], ts[j], descending=True)
            c = jnp.maximum(s, lax.rev(b, (0,)))
            s, _ = plsc.sort_key_val(c, c, descending=True)

        @pl.when(wid == 0)
        def _pos():
            tgt_v[...] = tvec
            pltpu.async_copy(rs_hbm.at[tgt_v], pos_v, sem).wait()

        with jax.named_scope("hnl_pub"):
            tloc_v[...] = s
            pltpu.sync_copy(tloc_v, shared.at[pl.ds(wid * L, L)])
            plsc.subcore_barrier()

        @pl.when(wid == 0)
        def _merge():
          with jax.named_scope("hnl_merge"):
            pltpu.sync_copy(shared, cand_v)
            m = cand_v[pl.ds(0, L)]
            for k in range(1, NW):
                b = cand_v[pl.ds(k * L, L)]
                c = jnp.maximum(m, lax.rev(b, (0,)))
                m, _ = plsc.sort_key_val(c, c, descending=True)

            lane = lax.iota(jnp.int32, L)
            f = jnp.maximum(MARGIN_ - pos_v[...] + m, 0.0)
            sel = jnp.where(lane < 5, f, 0.0)
            loss = jnp.sum(sel) * (1.0 / 5.0)
            out_v[...] = jnp.broadcast_to(loss, (L,))
            pltpu.sync_copy(out_v, out_hbm)


_hnl = functools.partial(
    pl.kernel,
    out_type=jax.ShapeDtypeStruct((L,), jnp.float32),
    mesh=plsc.VectorSubcoreMesh(core_axis_name="c", subcore_axis_name="s",
                                num_cores=1),
    compiler_params=pltpu.CompilerParams(needs_layout_passes=False,
                                         disable_bounds_checks=True,
                                         disable_semaphore_checks=True),
    scratch_types=[
        pltpu.VMEM((CHUNK,), jnp.float32),
        pltpu.VMEM((80,), jnp.int32),
        pltpu.VMEM((L,), jnp.int32),
        pltpu.VMEM((L,), jnp.float32),
        pltpu.VMEM((L,), jnp.float32),
        pltpu.VMEM((NW * L,), jnp.float32),
        pltpu.VMEM((L,), jnp.float32),
        pltpu.VMEM_SHARED((NW * L,), jnp.float32),
        pltpu.SemaphoreType.DMA,
        pltpu.SemaphoreType.DMA,
        pltpu.SemaphoreType.DMA,
        pltpu.SemaphoreType.DMA,
        pltpu.SemaphoreType.DMA,
        pltpu.SemaphoreType.DMA,
    ],
)(_hnl_body)


def kernel(rule_scores, target_rule, available_rules):
    tgt = jnp.asarray(target_rule, jnp.int32)
    forb = jnp.concatenate(
        [jnp.broadcast_to(tgt, (L,)), available_rules.astype(jnp.int32)])
    out16 = _hnl(rule_scores, forb)
    return out16[0]
